# baseline (speedup 1.0000x reference)
import functools
import numpy as np
import jax
import jax.numpy as jnp
from jax import lax
from jax.experimental import pallas as pl
from jax.experimental.pallas import tpu as pltpu
from jax.experimental.pallas import tpu_sc as plsc

_deg_hist = np.array([0.0] * 16 + [10000.0], dtype=np.float64)
_bins = np.arange(_deg_hist.size, dtype=np.float64)
_AVG_LOG = float((np.log(_bins + 1.0) * _deg_hist).sum() / _deg_hist.sum())

_SB = 2000
_LANES = 16


def _pre_body(x_ref, wt_ref, wb_ref, bias_ref, a_ref, b_ref):
    x = x_ref[...]
    a_ref[...] = jnp.dot(x, wt_ref[...], preferred_element_type=jnp.float32) + bias_ref[...]
    b_ref[...] = jnp.dot(x, wb_ref[...], preferred_element_type=jnp.float32)


def _pre(x, wtop, wbot, bias, blk):
    n, din = x.shape
    d = wtop.shape[1]
    grid = (n // blk,)
    return pl.pallas_call(
        _pre_body,
        grid=grid,
        in_specs=[
            pl.BlockSpec((blk, din), lambda i: (i, 0)),
            pl.BlockSpec((din, d), lambda i: (0, 0)),
            pl.BlockSpec((din, d), lambda i: (0, 0)),
            pl.BlockSpec((d,), lambda i: (0,)),
        ],
        out_specs=[
            pl.BlockSpec((blk, d), lambda i: (i, 0)),
            pl.BlockSpec((blk, d), lambda i: (i, 0)),
        ],
        out_shape=[
            jax.ShapeDtypeStruct((n, d), jnp.float32),
            jax.ShapeDtypeStruct((n, d), jnp.float32),
        ],
    )(x, wtop, wbot, bias)


def _make_seg(n, e, dp):
    avail = 131071 - (2 * _SB + 2 * (_SB + _LANES) + 2 * _LANES * dp + 1024)
    cmax = avail // (3 * dp + _LANES)
    sweeps = -(-n // (32 * cmax))
    c = -(-n // (32 * sweeps))
    c = -(-c // 8) * 8
    npad = 32 * sweeps * c
    nblocks = e // _SB
    assert e % _SB == 0
    mesh = plsc.VectorSubcoreMesh(core_axis_name="c", subcore_axis_name="s")
    osds = jax.ShapeDtypeStruct((npad * dp,), jnp.float32)
    dsds = jax.ShapeDtypeStruct((npad * _LANES,), jnp.float32)

    @functools.partial(
        pl.kernel,
        out_type=(osds, osds, osds, dsds),
        mesh=mesh,
        scratch_types=[
            pltpu.VMEM((_SB,), jnp.int32),
            pltpu.VMEM((_SB,), jnp.int32),
            pltpu.VMEM((_SB + _LANES,), jnp.int32),
            pltpu.VMEM((_SB + _LANES,), jnp.int32),
            pltpu.VMEM((_LANES, dp), jnp.float32),
            pltpu.VMEM((_LANES, dp), jnp.float32),
            pltpu.VMEM((c * dp,), jnp.float32),
            pltpu.VMEM((c * dp,), jnp.float32),
            pltpu.VMEM((c * dp,), jnp.float32),
            pltpu.VMEM((c * _LANES,), jnp.float32),
            pltpu.VMEM((_LANES,), jnp.int32),
            pltpu.VMEM((_LANES,), jnp.int32),
            pltpu.SemaphoreType.DMA,
            pltpu.SemaphoreType.DMA,
        ],
    )
    def seg(a_hbm, b_hbm, dst_hbm, src_hbm, out_s, out_q, out_m, out_d,
            dbuf, sbuf, mdst, msrc, arows, brows, acc_s, acc_q, acc_m, acc_d,
            idxa, idxb, sem_a, sem_b):
        wid = lax.axis_index("s") * 2 + lax.axis_index("c")
        zi = jnp.zeros((_LANES,), jnp.int32)
        zf = jnp.zeros((_LANES,), jnp.float32)
        of = jnp.ones((_LANES,), jnp.float32)
        ninf = jnp.full((_LANES,), -1e30, jnp.float32)

        def initm(i, _):
            mdst[pl.ds(i * _LANES, _LANES)] = zi
            msrc[pl.ds(i * _LANES, _LANES)] = zi
            return 0
        lax.fori_loop(0, (_SB + _LANES) // _LANES, initm, 0)

        def sweep_body(s, _):
            lo = (s * 32 + wid) * c
            hi = lo + c

            def zbody(i, _):
                acc_s[pl.ds(i * _LANES, _LANES)] = zf
                acc_q[pl.ds(i * _LANES, _LANES)] = zf
                acc_m[pl.ds(i * _LANES, _LANES)] = ninf
                return 0
            lax.fori_loop(0, (c * dp) // _LANES, zbody, 0)

            def zd(i, _):
                acc_d[pl.ds(i * _LANES, _LANES)] = zf
                return 0
            lax.fori_loop(0, c, zd, 0)

            def block_body(b, _):
                pltpu.sync_copy(dst_hbm.at[pl.ds(b * _SB, _SB)], dbuf)
                pltpu.sync_copy(src_hbm.at[pl.ds(b * _SB, _SB)], sbuf)

                def scan_body(v, m):
                    dv = dbuf[pl.ds(v * _LANES, _LANES)]
                    sv = sbuf[pl.ds(v * _LANES, _LANES)]
                    mm = m
                    for j in range(_LANES):
                        cond = (dv[j] >= lo) & (dv[j] < hi)

                        @pl.when(cond)
                        def _app(j=j, mm=mm):
                            mdst[pl.ds(mm, _LANES)] = jnp.full(
                                (_LANES,), dv[j], jnp.int32)
                            msrc[pl.ds(mm, _LANES)] = jnp.full(
                                (_LANES,), sv[j], jnp.int32)
                            doff = (dv[j] - lo) * _LANES
                            acc_d[pl.ds(doff, _LANES)] = (
                                acc_d[pl.ds(doff, _LANES)] + of)
                        mm = mm + jnp.where(cond, 1, 0)
                    return mm
                m = lax.fori_loop(0, _SB // _LANES, scan_body, 0)

                def group_body(g, _):
                    dl = mdst[pl.ds(g * _LANES, _LANES)]
                    idxa[...] = dl
                    idxb[...] = msrc[pl.ds(g * _LANES, _LANES)]
                    cpa = pltpu.async_copy(a_hbm.at[idxa], arows, sem_a)
                    cpb = pltpu.async_copy(b_hbm.at[idxb], brows, sem_b)
                    cpa.wait()
                    cpb.wait()
                    for j in range(_LANES):
                        @pl.when(g * _LANES + j < m)
                        def _do(j=j):
                            base = (dl[j] - lo) * dp

                            def col_body(cc, _):
                                off = base + cc * _LANES
                                h = (arows[j, pl.ds(cc * _LANES, _LANES)]
                                     + brows[j, pl.ds(cc * _LANES, _LANES)])
                                acc_s[pl.ds(off, _LANES)] = acc_s[pl.ds(off, _LANES)] + h
                                acc_q[pl.ds(off, _LANES)] = acc_q[pl.ds(off, _LANES)] + h * h
                                acc_m[pl.ds(off, _LANES)] = jnp.maximum(
                                    acc_m[pl.ds(off, _LANES)], h)
                                return 0
                            lax.fori_loop(0, dp // _LANES, col_body, 0)
                    return 0
                lax.fori_loop(0, (m + _LANES - 1) // _LANES, group_body, 0)
                return 0
            lax.fori_loop(0, nblocks, block_body, 0)

            pltpu.sync_copy(acc_s, out_s.at[pl.ds(lo * dp, c * dp)])
            pltpu.sync_copy(acc_q, out_q.at[pl.ds(lo * dp, c * dp)])
            pltpu.sync_copy(acc_m, out_m.at[pl.ds(lo * dp, c * dp)])
            pltpu.sync_copy(acc_d, out_d.at[pl.ds(lo * _LANES, c * _LANES)])
            return 0
        lax.fori_loop(0, sweeps, sweep_body, 0)

    return seg, npad


def _post_body(x_ref, s_ref, q_ref, m_ref, d_ref, wx_ref, wa_ref, wb_ref,
               wc_ref, bpost_ref, wlin_ref, blin_ref, g_ref, beta_ref, o_ref):
    cnt = d_ref[:, 0:1]
    cntc = jnp.maximum(cnt, 1.0)
    ssum = s_ref[...]
    mean = ssum / cntc
    mean2 = q_ref[...] / cntc
    var = jax.nn.relu(mean2 - mean * mean)
    std = jnp.sqrt(var + 1e-5)
    mx = jnp.where(cnt > 0, m_ref[...], 0.0)
    agg = jnp.concatenate([mean, ssum, mx, std], axis=1)
    logd = jnp.log(cntc + 1.0)
    sa = logd / _AVG_LOG
    sb = _AVG_LOG / logd
    z = (jnp.dot(x_ref[...], wx_ref[...], preferred_element_type=jnp.float32)
         + jnp.dot(agg, wa_ref[...], preferred_element_type=jnp.float32))
    zb = jnp.dot(agg, wb_ref[...], preferred_element_type=jnp.float32)
    zc = jnp.dot(agg, wc_ref[...], preferred_element_type=jnp.float32)
    out = z + sa * zb + sb * zc + bpost_ref[...]
    out = jnp.dot(out, wlin_ref[...], preferred_element_type=jnp.float32) + blin_ref[...]
    mu = jnp.mean(out, axis=-1, keepdims=True)
    vv = jnp.mean((out - mu) ** 2, axis=-1, keepdims=True)
    ln = (out - mu) / jnp.sqrt(vv + 1e-5) * g_ref[...] + beta_ref[...]
    o_ref[...] = jnp.where(ln > 0, ln, jnp.exp(jnp.minimum(ln, 0.0)) - 1.0)


def _post(x, ssum, sq, mx, dg, wpost, bpost, wlin, blin, g, beta, d, blk):
    n, din = x.shape
    wx = wpost[:din]
    wa = wpost[din:din + 4 * d]
    wb = wpost[din + 4 * d:din + 8 * d]
    wc = wpost[din + 8 * d:]
    grid = (n // blk,)
    full = lambda arr: pl.BlockSpec(arr.shape, lambda i: (0,) * arr.ndim)
    return pl.pallas_call(
        _post_body,
        grid=grid,
        in_specs=[
            pl.BlockSpec((blk, din), lambda i: (i, 0)),
            pl.BlockSpec((blk, d), lambda i: (i, 0)),
            pl.BlockSpec((blk, d), lambda i: (i, 0)),
            pl.BlockSpec((blk, d), lambda i: (i, 0)),
            pl.BlockSpec((blk, _LANES), lambda i: (i, 0)),
            full(wx), full(wa), full(wb), full(wc), full(bpost),
            full(wlin), full(blin), full(g), full(beta),
        ],
        out_specs=pl.BlockSpec((blk, wlin.shape[1]), lambda i: (i, 0)),
        out_shape=jax.ShapeDtypeStruct((n, wlin.shape[1]), jnp.float32),
    )(x, ssum, sq, mx, dg, wx, wa, wb, wc, bpost, wlin, blin, g, beta)


def _final_body(h_ref, w_ref, b_ref, g_ref, beta_ref, o_ref):
    phi = jnp.dot(h_ref[...], w_ref[...], preferred_element_type=jnp.float32) + b_ref[...]
    mu = jnp.mean(phi, axis=-1, keepdims=True)
    var = jnp.mean((phi - mu) ** 2, axis=-1, keepdims=True)
    o_ref[...] = (phi - mu) / jnp.sqrt(var + 1e-5) * g_ref[...] + beta_ref[...]


def _final(h, wout, bout, g, beta, blk):
    n = h.shape[0]
    grid = (n // blk,)
    return pl.pallas_call(
        _final_body,
        grid=grid,
        in_specs=[
            pl.BlockSpec((blk, h.shape[1]), lambda i: (i, 0)),
            pl.BlockSpec(wout.shape, lambda i: (0, 0)),
            pl.BlockSpec(bout.shape, lambda i: (0,)),
            pl.BlockSpec(g.shape, lambda i: (0,)),
            pl.BlockSpec(beta.shape, lambda i: (0,)),
        ],
        out_specs=pl.BlockSpec((blk, wout.shape[1]), lambda i: (i, 0)),
        out_shape=jax.ShapeDtypeStruct((n, wout.shape[1]), jnp.float32),
    )(h, wout, bout, g, beta)


def _layer(x, dst, src, wpre, bpre, wpost, bpost, wlin, blin, g, beta, blk):
    n, din = x.shape
    d = wpre.shape[1]
    a, b = _pre(x, wpre[:din], wpre[din:], bpre, blk)
    seg, npad = _make_seg(n, dst.shape[0], d)
    ssum, sq, mx, dg = seg(a, b, dst, src)
    ssum = ssum.reshape(npad, d)[:n]
    sq = sq.reshape(npad, d)[:n]
    mx = mx.reshape(npad, d)[:n]
    dg = dg.reshape(npad, _LANES)[:n]
    return _post(x, ssum, sq, mx, dg, wpost, bpost, wlin, blin, g, beta, d, blk)


def kernel(x, edge_index, Wpre1, bpre1, Wpost1, bpost1, Wlin1, blin1, g1, b1,
           Wpre2, bpre2, Wpost2, bpost2, Wlin2, blin2, g2, b2, Wout, bout, go, bo):
    src = edge_index[0]
    dst = edge_index[1]
    blk = 1000 if x.shape[0] % 1000 == 0 else 8
    h = _layer(x, dst, src, Wpre1, bpre1, Wpost1, bpost1, Wlin1, blin1, g1, b1, blk)
    h = _layer(h, dst, src, Wpre2, bpre2, Wpost2, bpost2, Wlin2, blin2, g2, b2, blk)
    return _final(h, Wout, bout, go, bo, blk)
